# baseline (speedup 1.0000x reference)
import functools

import jax
import jax.numpy as jnp
from jax import lax
from jax.experimental import pallas as pl
from jax.experimental.pallas import tpu as pltpu
from jax.experimental.pallas import tpu_sc as plsc

_N_EMB = 1024
_D = 64
_TOK = 32 * 576
_BLK = 4608
_NBLK = _TOK // _BLK


def _argmin_body(ze_ref, cb_ref, rhs_ref, am_ref, ent_ref, loss_ref,
                 cbp_ref, counts_ref, acc_ref, b_ref, cbt_ref):
    i = pl.program_id(0)
    ze = ze_ref[...]
    a = jnp.sum(ze * ze, axis=1, keepdims=True)

    @pl.when(i == 0)
    def _init_b():
        cb = cb_ref[...]
        cbt_ref[...] = cb.T
        b_ref[...] = jnp.sum(cbt_ref[...] * cbt_ref[...], axis=0,
                             keepdims=True)
        cbp_ref[:, 0:_D] = cb
        cbp_ref[:, _D:2 * _D] = cb

    cbt = cbt_ref[...]
    b = b_ref[...]
    mmn = lax.dot_general(ze * (-2.0), cbt, (((1,), (0,)), ((), ())),
                          preferred_element_type=jnp.float32)
    sq = (a + b) + mmn
    m = jnp.min(sq, axis=1)
    eqf = jnp.where(sq == m[:, None], 1.0, 0.0).astype(jnp.bfloat16)
    mix = lax.dot_general(eqf, rhs_ref[...], (((1,), (0,)), ((), ())),
                          preferred_element_type=jnp.float32)
    mix_t = mix.T
    am0 = (mix_t[0:1, :] + mix_t[1:2, :]).astype(jnp.int32)
    cnt = mix_t[2:3, :]
    ones_row = jnp.ones((1, _BLK), jnp.bfloat16)
    counts_blk = lax.dot_general(ones_row, eqf, (((1,), (0,)), ((), ())),
                                 preferred_element_type=jnp.float32)
    am_ref[...] = am0.reshape(1, _BLK // 128, 128)

    @pl.when(i == 0)
    def _init():
        counts_ref[...] = jnp.zeros_like(counts_ref)
        acc_ref[0] = 0.0

    counts_ref[...] += counts_blk
    acc_ref[0] += jnp.sum(m)

    @pl.when(jnp.max(cnt) > 1.5)
    def _ties():
        iota = lax.broadcasted_iota(jnp.int32, (_BLK, _N_EMB), 1)
        am = jnp.min(jnp.where(sq == m[:, None], iota, _N_EMB), axis=1)
        am_ref[...] = am.reshape(1, _BLK // 128, 128)
        onehot = (iota == am[:, None]).astype(jnp.float32)
        counts_ref[...] += jnp.sum(onehot, axis=0, keepdims=True) - counts_blk

    @pl.when(i == _NBLK - 1)
    def _finish():
        probs = counts_ref[0, :] / 10.0
        ent_ref[...] = jnp.sum(probs * jnp.log(probs + 1e-10)).reshape(1, 1)
        loss_ref[...] = (acc_ref[0] / float(_TOK * _D)).reshape(1, 1)


def _mix_rhs():
    idx = jnp.arange(_N_EMB, dtype=jnp.int32)
    rhs = jnp.zeros((_N_EMB, 128), jnp.bfloat16)
    rhs = rhs.at[:, 0].set(((idx // 8) * 8).astype(jnp.bfloat16))
    rhs = rhs.at[:, 1].set((idx % 8).astype(jnp.bfloat16))
    rhs = rhs.at[:, 2].set(jnp.bfloat16(1))
    return rhs


def _argmin_losses(ze2d, cb):
    return pl.pallas_call(
        _argmin_body,
        grid=(_NBLK,),
        in_specs=[
            pl.BlockSpec((_BLK, _D), lambda i: (i, 0)),
            pl.BlockSpec((_N_EMB, _D), lambda i: (0, 0)),
            pl.BlockSpec((_N_EMB, 128), lambda i: (0, 0)),
        ],
        out_specs=[
            pl.BlockSpec((1, _BLK // 128, 128), lambda i: (i, 0, 0)),
            pl.BlockSpec((1, 1), lambda i: (0, 0)),
            pl.BlockSpec((1, 1), lambda i: (0, 0)),
            pl.BlockSpec((_N_EMB, 2 * _D), lambda i: (0, 0)),
        ],
        out_shape=[
            jax.ShapeDtypeStruct((_NBLK, _BLK // 128, 128), jnp.int32),
            jax.ShapeDtypeStruct((1, 1), jnp.float32),
            jax.ShapeDtypeStruct((1, 1), jnp.float32),
            jax.ShapeDtypeStruct((_N_EMB, 2 * _D), jnp.float32),
        ],
        scratch_shapes=[
            pltpu.VMEM((1, _N_EMB), jnp.float32),
            pltpu.SMEM((1,), jnp.float32),
            pltpu.VMEM((1, _N_EMB), jnp.float32),
            pltpu.VMEM((_D, _N_EMB), jnp.float32),
        ],
    )(ze2d, cb, _mix_rhs())


def _sc_gather_half(cb_pad, idx2d):
    mesh = plsc.VectorSubcoreMesh(core_axis_name="c", subcore_axis_name="s")

    @functools.partial(
        pl.kernel,
        out_type=jax.ShapeDtypeStruct((_TOK // 2, 2 * _D), jnp.float32),
        mesh=mesh,
        scratch_types=[
            pltpu.VMEM((16, 128), jnp.int32),
            pltpu.VMEM((3 * 128, 2 * _D), jnp.float32),
            pltpu.SemaphoreType.DMA,
        ],
    )
    def gather_kernel(cb_hbm, idx_hbm, out_hbm, idx_v, rows_v, sem):
        wid = lax.axis_index("s") * 2 + lax.axis_index("c")
        base = jnp.where(wid < 8, 3 * wid, 2 * wid + 8)
        astart = pl.multiple_of(jnp.minimum((base // 8) * 8, 72 - 16), 8)
        off = base - astart
        pltpu.sync_copy(idx_hbm.at[pl.ds(astart, 16)], idx_v)
        copies = [
            pltpu.async_copy(cb_hbm.at[idx_v.at[off + j]],
                             rows_v.at[pl.ds(j * 128, 128)], sem)
            for j in range(2)
        ]

        @pl.when(wid < 8)
        def _three_chunks():
            c3 = pltpu.async_copy(cb_hbm.at[idx_v.at[off + 2]],
                                  rows_v.at[pl.ds(256, 128)], sem)
            for c in copies:
                c.wait()
            c3.wait()
            pltpu.sync_copy(rows_v,
                            out_hbm.at[pl.ds(base * 128, 3 * 128)])

        @pl.when(wid >= 8)
        def _two_chunks():
            for c in copies:
                c.wait()
            pltpu.sync_copy(rows_v.at[pl.ds(0, 256)],
                            out_hbm.at[pl.ds(base * 128, 2 * 128)])

    return gather_kernel(cb_pad, idx2d)


def kernel(ze, codebook):
    ze2d = ze.reshape(_TOK, _D)
    am3d, ent, loss, cb_pad = _argmin_losses(ze2d, codebook)
    idx2d = am3d.reshape(_TOK // 128, 128)
    z1 = _sc_gather_half(cb_pad, idx2d[:_TOK // 256])
    z2 = _sc_gather_half(cb_pad, idx2d[_TOK // 256:])
    zq = jnp.concatenate([z1[:, :_D], z2[:, :_D]], axis=0)
    argmin = am3d.reshape(ze.shape[0], ze.shape[1])
    vq_e_loss = loss[0, 0]
    return (argmin, zq.reshape(ze.shape), vq_e_loss, vq_e_loss, ent[0, 0])


# speedup vs baseline: 1.1563x; 1.1563x over previous
import functools

import jax
import jax.numpy as jnp
from jax import lax
from jax.experimental import pallas as pl
from jax.experimental.pallas import tpu as pltpu
from jax.experimental.pallas import tpu_sc as plsc

_N_EMB = 1024
_D = 64
_TOK = 32 * 576
_BLK = 4608
_NBLK = _TOK // _BLK


def _argmin_body(ze_ref, cb_ref, rhs_ref, am_ref, ent_ref, loss_ref,
                 cbp_ref, counts_ref, acc_ref, b_ref, cbt_ref):
    i = pl.program_id(0)
    ze = ze_ref[...]
    a = jnp.sum(ze * ze, axis=1, keepdims=True)

    @pl.when(i == 0)
    def _init_b():
        cb = cb_ref[...]
        cbt_ref[...] = cb.T
        b_ref[...] = jnp.sum(cbt_ref[...] * cbt_ref[...], axis=0,
                             keepdims=True)
        cbp_ref[:, 0:_D] = cb
        cbp_ref[:, _D:2 * _D] = cb

    cbt = cbt_ref[...]
    b = b_ref[...]
    mmn = lax.dot_general(ze * (-2.0), cbt, (((1,), (0,)), ((), ())),
                          preferred_element_type=jnp.float32)
    sq = (a + b) + mmn
    m = jnp.min(sq, axis=1)
    eqf = jnp.where(sq == m[:, None], 1.0, 0.0).astype(jnp.bfloat16)
    mix = lax.dot_general(eqf, rhs_ref[...], (((1,), (0,)), ((), ())),
                          preferred_element_type=jnp.float32)
    mix_t = mix.T
    am0 = (mix_t[0:1, :] + mix_t[1:2, :]).astype(jnp.int32)
    cnt = mix_t[2:3, :]
    ones_row = jnp.ones((1, _BLK), jnp.bfloat16)
    counts_blk = lax.dot_general(ones_row, eqf, (((1,), (0,)), ((), ())),
                                 preferred_element_type=jnp.float32)
    am_ref[...] = am0.reshape(1, _BLK // 128, 128)

    @pl.when(i == 0)
    def _init():
        counts_ref[...] = jnp.zeros_like(counts_ref)
        acc_ref[0] = 0.0

    counts_ref[...] += counts_blk
    acc_ref[0] += jnp.sum(m)

    @pl.when(jnp.max(cnt) > 1.5)
    def _ties():
        iota = lax.broadcasted_iota(jnp.int32, (_BLK, _N_EMB), 1)
        am = jnp.min(jnp.where(sq == m[:, None], iota, _N_EMB), axis=1)
        am_ref[...] = am.reshape(1, _BLK // 128, 128)
        onehot = (iota == am[:, None]).astype(jnp.float32)
        counts_ref[...] += jnp.sum(onehot, axis=0, keepdims=True) - counts_blk

    @pl.when(i == _NBLK - 1)
    def _finish():
        probs = counts_ref[0, :] / 10.0
        ent_ref[...] = jnp.sum(probs * jnp.log(probs + 1e-10)).reshape(1, 1)
        loss_ref[...] = (acc_ref[0] / float(_TOK * _D)).reshape(1, 1)


def _mix_rhs():
    idx = jnp.arange(_N_EMB, dtype=jnp.int32)
    rhs = jnp.zeros((_N_EMB, 128), jnp.bfloat16)
    rhs = rhs.at[:, 0].set(((idx // 8) * 8).astype(jnp.bfloat16))
    rhs = rhs.at[:, 1].set((idx % 8).astype(jnp.bfloat16))
    rhs = rhs.at[:, 2].set(jnp.bfloat16(1))
    return rhs


def _argmin_losses(ze2d, cb):
    return pl.pallas_call(
        _argmin_body,
        grid=(_NBLK,),
        in_specs=[
            pl.BlockSpec((_BLK, _D), lambda i: (i, 0)),
            pl.BlockSpec((_N_EMB, _D), lambda i: (0, 0)),
            pl.BlockSpec((_N_EMB, 128), lambda i: (0, 0)),
        ],
        out_specs=[
            pl.BlockSpec((1, _BLK // 128, 128), lambda i: (i, 0, 0)),
            pl.BlockSpec((1, 1), lambda i: (0, 0)),
            pl.BlockSpec((1, 1), lambda i: (0, 0)),
            pl.BlockSpec((_N_EMB, 2 * _D), lambda i: (0, 0)),
        ],
        out_shape=[
            jax.ShapeDtypeStruct((_NBLK, _BLK // 128, 128), jnp.int32),
            jax.ShapeDtypeStruct((1, 1), jnp.float32),
            jax.ShapeDtypeStruct((1, 1), jnp.float32),
            jax.ShapeDtypeStruct((_N_EMB, 2 * _D), jnp.float32),
        ],
        scratch_shapes=[
            pltpu.VMEM((1, _N_EMB), jnp.float32),
            pltpu.SMEM((1,), jnp.float32),
            pltpu.VMEM((1, _N_EMB), jnp.float32),
            pltpu.VMEM((_D, _N_EMB), jnp.float32),
        ],
    )(ze2d, cb, _mix_rhs())


def _sc_gather(cb_pad, idx2d):
    mesh = plsc.VectorSubcoreMesh(core_axis_name="c", subcore_axis_name="s")

    @functools.partial(
        pl.kernel,
        out_type=jax.ShapeDtypeStruct((_TOK, 2 * _D), jnp.float32),
        mesh=mesh,
        scratch_types=[
            pltpu.VMEM((32, 128), jnp.int32),
            pltpu.VMEM((5 * 128, 2 * _D), jnp.float32),
            pltpu.SemaphoreType.DMA,
        ],
    )
    def gather_kernel(cb_hbm, idx_hbm, out_hbm, idx_v, rows_v, sem):
        wid = lax.axis_index("s") * 2 + lax.axis_index("c")
        base = jnp.where(wid < 16, 5 * wid, 4 * wid + 16)
        astart = pl.multiple_of(jnp.minimum((base // 8) * 8, 144 - 32), 8)
        off = base - astart
        pltpu.sync_copy(idx_hbm.at[pl.ds(astart, 32)], idx_v)
        copies = [
            pltpu.async_copy(cb_hbm.at[idx_v.at[off + j]],
                             rows_v.at[pl.ds(j * 128, 128)], sem)
            for j in range(4)
        ]

        @pl.when(wid < 16)
        def _fifth_chunk():
            c5 = pltpu.async_copy(cb_hbm.at[idx_v.at[off + 4]],
                                  rows_v.at[pl.ds(512, 128)], sem)
            for c in copies:
                c.wait()
            c5.wait()
            pltpu.sync_copy(rows_v,
                            out_hbm.at[pl.ds(base * 128, 5 * 128)])

        @pl.when(wid >= 16)
        def _four_chunks():
            for c in copies:
                c.wait()
            pltpu.sync_copy(rows_v.at[pl.ds(0, 512)],
                            out_hbm.at[pl.ds(base * 128, 4 * 128)])

    return gather_kernel(cb_pad, idx2d)


def kernel(ze, codebook):
    ze2d = ze.reshape(_TOK, _D)
    am3d, ent, loss, cb_pad = _argmin_losses(ze2d, codebook)
    zq = _sc_gather(cb_pad, am3d.reshape(_TOK // 128, 128))[:, :_D]
    argmin = am3d.reshape(ze.shape[0], ze.shape[1])
    vq_e_loss = loss[0, 0]
    return (argmin, zq.reshape(ze.shape), vq_e_loss, vq_e_loss, ent[0, 0])
